# baseline (speedup 1.0000x reference)
import functools

import jax
import jax.numpy as jnp
import numpy as np
from jax import lax
from jax.experimental import pallas as pl
from jax.experimental.pallas import tpu as pltpu
from jax.experimental.pallas import tpu_sc as plsc

B, T, D = 4, 2048, 768
NV = 8
NR = 4
NTOK = B * T
TB = 256
TBLK = T // TB
NW = 32
TPW = NTOK // NW
CH = 16


def _sinusoidal_pe(seq_len, d_model):
    pos = np.arange(seq_len, dtype=np.float32)[:, None]
    div = np.exp(np.arange(0, d_model, 2, dtype=np.float32) * (-np.log(10000.0) / d_model))
    pe = np.zeros((seq_len, d_model), dtype=np.float32)
    pe[:, 0::2] = np.sin(pos * div)
    pe[:, 1::2] = np.cos(pos * div)
    return pe


def _sc_rank_kernel(noise_hbm, masked_hbm, revert_hbm, nz_v, mk_v, rv_v):
    wid = lax.axis_index("s") * 2 + lax.axis_index("c")
    tok0 = wid * TPW
    pltpu.sync_copy(noise_hbm.at[pl.ds(tok0 * NV, TPW * NV)], nz_v)
    ones = jnp.full((CH,), 1, jnp.int32)
    zeros = jnp.full((CH,), 0, jnp.int32)
    lane = lax.iota(jnp.int32, CH)
    def _chunk(c, _):
        base = (lane + jnp.full((CH,), c * CH, jnp.int32)) * jnp.full((CH,), NV, jnp.int32)
        nv = [plsc.load_gather(nz_v, [base + jnp.full((CH,), i, jnp.int32)])
              for i in range(NV)]
        tokrel = lane + jnp.full((CH,), c * CH, jnp.int32)
        ranks = []
        for i in range(NV):
            r = zeros
            for j in range(NV):
                if j < i:
                    cij = nv[j] <= nv[i]
                elif j > i:
                    cij = nv[j] < nv[i]
                else:
                    continue
                r = r + jnp.where(cij, ones, zeros)
            ranks.append(r)
        nvv = jnp.full((CH,), NV, jnp.int32)
        for i in range(NV):
            plsc.store_scatter(rv_v, [tokrel * nvv + jnp.full((CH,), i, jnp.int32)],
                               ranks[i])
        nmk = jnp.full((CH,), NV - NR, jnp.int32)
        for k in range(NR, NV):
            s_k = zeros
            kv = jnp.full((CH,), k, jnp.int32)
            for i in range(NV):
                iv = jnp.full((CH,), i, jnp.int32)
                s_k = s_k + jnp.where(ranks[i] == kv, iv, zeros)
            plsc.store_scatter(mk_v, [tokrel * nmk + jnp.full((CH,), k - NR, jnp.int32)],
                               s_k)
        return 0

    lax.fori_loop(0, TPW // CH, _chunk, 0)
    pltpu.sync_copy(mk_v, masked_hbm.at[pl.ds(tok0 * (NV - NR), TPW * (NV - NR))])
    pltpu.sync_copy(rv_v, revert_hbm.at[pl.ds(tok0 * NV, TPW * NV)])


_sc_rank = functools.partial(
    pl.kernel,
    mesh=plsc.VectorSubcoreMesh(core_axis_name="c", subcore_axis_name="s"),
    compiler_params=pltpu.CompilerParams(needs_layout_passes=False),
    out_type=[
        jax.ShapeDtypeStruct((NTOK * (NV - NR),), jnp.int32),
        jax.ShapeDtypeStruct((NTOK * NV,), jnp.int32),
    ],
    scratch_types=[
        pltpu.VMEM((TPW * NV,), jnp.float32),
        pltpu.VMEM((TPW * (NV - NR),), jnp.int32),
        pltpu.VMEM((TPW * NV,), jnp.int32),
    ],
)(_sc_rank_kernel)


def _block_remain_kernel(g_ref, v0, v1, v2, v3, v4, v5, v6, v7,
                         noise_ref, emb_ref, pe_ref, out_ref):
    valid = [v0, v1, v2, v3, v4, v5, v6, v7]
    n = noise_ref[0]
    j_iota = jax.lax.broadcasted_iota(jnp.int32, (1, NV), 1)

    ranks = jnp.zeros((TB, NV), jnp.int32)
    for i in range(NV):
        ni = n[:, i:i + 1]
        lt = (n < ni)
        eq = (n == ni) & (j_iota < i)
        rank_i = jnp.sum((lt | eq).astype(jnp.int32), axis=1, keepdims=True)
        ranks = ranks + rank_i * (j_iota == i).astype(jnp.int32)

    shuffle = jnp.zeros((TB, NV), jnp.int32)
    for i in range(NV):
        ri = ranks[:, i:i + 1]
        shuffle = shuffle + jnp.where(ri == j_iota, i, 0)

    pe = pe_ref[pl.ds((pl.program_id(0) % TBLK) * TB, TB), :]
    out_ref[0, :, 0, :] = g_ref[0] + emb_ref[0:1, :] + pe

    vp = [valid[i][0] + emb_ref[i + 1:i + 2, :] for i in range(NV)]
    for k in range(NR):
        sel = shuffle[:, k:k + 1]
        acc = vp[0]
        for i in range(1, NV):
            acc = jnp.where(sel == i, vp[i], acc)
        out_ref[0, :, k + 1, :] = acc + pe


@jax.jit
def _run(g, vs, noise, noise_flat, emb16, pe):
    masked_f, revert_f = _sc_rank(noise_flat)
    tok_spec = pl.BlockSpec((1, TB, D), lambda i: (i // TBLK, i % TBLK, 0))
    out = pl.pallas_call(
        _block_remain_kernel,
        grid=(B * TBLK,),
        in_specs=[tok_spec] * (1 + NV) + [
            pl.BlockSpec((1, TB, NV), lambda i: (i // TBLK, i % TBLK, 0)),
            pl.BlockSpec((16, D), lambda i: (0, 0)),
            pl.BlockSpec((T, D), lambda i: (0, 0)),
        ],
        out_specs=pl.BlockSpec((1, TB, NR + 1, D), lambda i: (i // TBLK, i % TBLK, 0, 0)),
        out_shape=jax.ShapeDtypeStruct((B, T, NR + 1, D), jnp.float32),
    )(g, *vs, noise, emb16, pe)
    return (out,
            masked_f.reshape(B, T, NV - NR),
            revert_f.reshape(B, T, NV))


def kernel(data_global, data_t0, data_t1, data_t2, data_t3, data_t4,
           data_t5, data_t6, data_t7, noise, mod_emb):
    vs = [data_t0, data_t1, data_t2, data_t3, data_t4, data_t5, data_t6, data_t7]
    emb16 = jnp.zeros((16, D), jnp.float32).at[:NV + 1].set(mod_emb)
    pe = jnp.asarray(_sinusoidal_pe(T, D))
    noise_flat = noise.reshape(NTOK * NV)
    return _run(data_global, vs, noise, noise_flat, emb16, pe)


# speedup vs baseline: 1.0117x; 1.0117x over previous
import functools

import jax
import jax.numpy as jnp
import numpy as np
from jax import lax
from jax.experimental import pallas as pl
from jax.experimental.pallas import tpu as pltpu
from jax.experimental.pallas import tpu_sc as plsc

B, T, D = 4, 2048, 768
NV = 8
NR = 4
NTOK = B * T
TB = 256
TBLK = T // TB
NW = 32
TPW = NTOK // NW
CH = 16


def _sinusoidal_pe(seq_len, d_model):
    pos = np.arange(seq_len, dtype=np.float32)[:, None]
    div = np.exp(np.arange(0, d_model, 2, dtype=np.float32) * (-np.log(10000.0) / d_model))
    pe = np.zeros((seq_len, d_model), dtype=np.float32)
    pe[:, 0::2] = np.sin(pos * div)
    pe[:, 1::2] = np.cos(pos * div)
    return pe


def _sc_rank_kernel(noise_hbm, masked_hbm, revert_hbm, nz_v, mk_v, rv_v):
    wid = lax.axis_index("s") * 2 + lax.axis_index("c")
    tok0 = wid * TPW
    pltpu.sync_copy(noise_hbm.at[pl.ds(tok0 * NV, TPW * NV)], nz_v)
    ones = jnp.full((CH,), 1, jnp.int32)
    zeros = jnp.full((CH,), 0, jnp.int32)
    lane = lax.iota(jnp.int32, CH)
    def _chunk(c, _):
        base = (lane + jnp.full((CH,), c * CH, jnp.int32)) * jnp.full((CH,), NV, jnp.int32)
        nv = [plsc.load_gather(nz_v, [base + jnp.full((CH,), i, jnp.int32)])
              for i in range(NV)]
        tokrel = lane + jnp.full((CH,), c * CH, jnp.int32)
        ranks = []
        for i in range(NV):
            r = zeros
            for j in range(NV):
                if j < i:
                    cij = nv[j] <= nv[i]
                elif j > i:
                    cij = nv[j] < nv[i]
                else:
                    continue
                r = r + jnp.where(cij, ones, zeros)
            ranks.append(r)
        for i in range(NV):
            plsc.store_scatter(rv_v, [tokrel, jnp.full((CH,), i, jnp.int32)],
                               ranks[i])
        for k in range(NR, NV):
            s_k = zeros
            kv = jnp.full((CH,), k, jnp.int32)
            for i in range(NV):
                iv = jnp.full((CH,), i, jnp.int32)
                s_k = s_k + jnp.where(ranks[i] == kv, iv, zeros)
            plsc.store_scatter(mk_v, [tokrel, jnp.full((CH,), k - NR, jnp.int32)],
                               s_k)
        return 0

    lax.fori_loop(0, TPW // CH, _chunk, 0)
    b0 = tok0 // T
    t0 = tok0 % T
    pltpu.sync_copy(mk_v, masked_hbm.at[b0, pl.ds(t0, TPW), :])
    pltpu.sync_copy(rv_v, revert_hbm.at[b0, pl.ds(t0, TPW), :])


_sc_rank = functools.partial(
    pl.kernel,
    mesh=plsc.VectorSubcoreMesh(core_axis_name="c", subcore_axis_name="s"),
    compiler_params=pltpu.CompilerParams(needs_layout_passes=False),
    out_type=[
        jax.ShapeDtypeStruct((B, T, NV - NR), jnp.int32),
        jax.ShapeDtypeStruct((B, T, NV), jnp.int32),
    ],
    scratch_types=[
        pltpu.VMEM((TPW * NV,), jnp.float32),
        pltpu.VMEM((TPW, NV - NR), jnp.int32),
        pltpu.VMEM((TPW, NV), jnp.int32),
    ],
)(_sc_rank_kernel)


def _block_remain_kernel(g_ref, v0, v1, v2, v3, v4, v5, v6, v7,
                         noise_ref, emb_ref, pe_ref, out_ref):
    valid = [v0, v1, v2, v3, v4, v5, v6, v7]
    n = noise_ref[0]
    j_iota = jax.lax.broadcasted_iota(jnp.int32, (1, NV), 1)

    ranks = jnp.zeros((TB, NV), jnp.int32)
    for i in range(NV):
        ni = n[:, i:i + 1]
        lt = (n < ni)
        eq = (n == ni) & (j_iota < i)
        rank_i = jnp.sum((lt | eq).astype(jnp.int32), axis=1, keepdims=True)
        ranks = ranks + rank_i * (j_iota == i).astype(jnp.int32)

    shuffle = jnp.zeros((TB, NV), jnp.int32)
    for i in range(NV):
        ri = ranks[:, i:i + 1]
        shuffle = shuffle + jnp.where(ri == j_iota, i, 0)

    pe = pe_ref[pl.ds((pl.program_id(0) % TBLK) * TB, TB), :]
    out_ref[0, :, 0, :] = g_ref[0] + emb_ref[0:1, :] + pe

    vp = [valid[i][0] + emb_ref[i + 1:i + 2, :] for i in range(NV)]
    for k in range(NR):
        sel = shuffle[:, k:k + 1]
        acc = vp[0]
        for i in range(1, NV):
            acc = jnp.where(sel == i, vp[i], acc)
        out_ref[0, :, k + 1, :] = acc + pe


@jax.jit
def _run(g, vs, noise, noise_flat, emb16, pe):
    masked_f, revert_f = _sc_rank(noise_flat)
    tok_spec = pl.BlockSpec((1, TB, D), lambda i: (i // TBLK, i % TBLK, 0))
    out = pl.pallas_call(
        _block_remain_kernel,
        grid=(B * TBLK,),
        in_specs=[tok_spec] * (1 + NV) + [
            pl.BlockSpec((1, TB, NV), lambda i: (i // TBLK, i % TBLK, 0)),
            pl.BlockSpec((16, D), lambda i: (0, 0)),
            pl.BlockSpec((T, D), lambda i: (0, 0)),
        ],
        out_specs=pl.BlockSpec((1, TB, NR + 1, D), lambda i: (i // TBLK, i % TBLK, 0, 0)),
        out_shape=jax.ShapeDtypeStruct((B, T, NR + 1, D), jnp.float32),
    )(g, *vs, noise, emb16, pe)
    return (out, masked_f, revert_f)


def kernel(data_global, data_t0, data_t1, data_t2, data_t3, data_t4,
           data_t5, data_t6, data_t7, noise, mod_emb):
    vs = [data_t0, data_t1, data_t2, data_t3, data_t4, data_t5, data_t6, data_t7]
    emb16 = jnp.zeros((16, D), jnp.float32).at[:NV + 1].set(mod_emb)
    pe = jnp.asarray(_sinusoidal_pe(T, D))
    noise_flat = noise.reshape(NTOK * NV)
    return _run(data_global, vs, noise, noise_flat, emb16, pe)


# speedup vs baseline: 1.0205x; 1.0086x over previous
import functools

import jax
import jax.numpy as jnp
import numpy as np
from jax import lax
from jax.experimental import pallas as pl
from jax.experimental.pallas import tpu as pltpu
from jax.experimental.pallas import tpu_sc as plsc

B, T, D = 4, 2048, 768
NV = 8
NR = 4
NTOK = B * T
TB = 256
TBLK = T // TB
NW = 32
TPW = NTOK // NW
CH = 16


def _sinusoidal_pe(seq_len, d_model):
    pos = np.arange(seq_len, dtype=np.float32)[:, None]
    div = np.exp(np.arange(0, d_model, 2, dtype=np.float32) * (-np.log(10000.0) / d_model))
    pe = np.zeros((seq_len, d_model), dtype=np.float32)
    pe[:, 0::2] = np.sin(pos * div)
    pe[:, 1::2] = np.cos(pos * div)
    return pe


def _sc_rank_kernel(noise_hbm, masked_hbm, revert_hbm, nz_v, mk_v, rv_v):
    wid = lax.axis_index("s") * 2 + lax.axis_index("c")
    tok0 = wid * TPW
    b0 = tok0 // T
    t0 = tok0 % T
    pltpu.sync_copy(noise_hbm.at[b0, pl.ds(t0, TPW), :], nz_v)
    ones = jnp.full((CH,), 1, jnp.int32)
    zeros = jnp.full((CH,), 0, jnp.int32)
    lane = lax.iota(jnp.int32, CH)
    def _chunk(c, _):
        tokrel = lane + jnp.full((CH,), c * CH, jnp.int32)
        nv = [plsc.load_gather(nz_v, [tokrel, jnp.full((CH,), i, jnp.int32)])
              for i in range(NV)]
        ranks = []
        for i in range(NV):
            r = zeros
            for j in range(NV):
                if j < i:
                    cij = nv[j] <= nv[i]
                elif j > i:
                    cij = nv[j] < nv[i]
                else:
                    continue
                r = r + jnp.where(cij, ones, zeros)
            ranks.append(r)
        for i in range(NV):
            plsc.store_scatter(rv_v, [tokrel, jnp.full((CH,), i, jnp.int32)],
                               ranks[i])
        for k in range(NR, NV):
            s_k = zeros
            kv = jnp.full((CH,), k, jnp.int32)
            for i in range(NV):
                iv = jnp.full((CH,), i, jnp.int32)
                s_k = s_k + jnp.where(ranks[i] == kv, iv, zeros)
            plsc.store_scatter(mk_v, [tokrel, jnp.full((CH,), k - NR, jnp.int32)],
                               s_k)
        return 0

    lax.fori_loop(0, TPW // CH, _chunk, 0)
    pltpu.sync_copy(mk_v, masked_hbm.at[b0, pl.ds(t0, TPW), :])
    pltpu.sync_copy(rv_v, revert_hbm.at[b0, pl.ds(t0, TPW), :])


_sc_rank = functools.partial(
    pl.kernel,
    mesh=plsc.VectorSubcoreMesh(core_axis_name="c", subcore_axis_name="s"),
    compiler_params=pltpu.CompilerParams(needs_layout_passes=False),
    out_type=[
        jax.ShapeDtypeStruct((B, T, NV - NR), jnp.int32),
        jax.ShapeDtypeStruct((B, T, NV), jnp.int32),
    ],
    scratch_types=[
        pltpu.VMEM((TPW, NV), jnp.float32),
        pltpu.VMEM((TPW, NV - NR), jnp.int32),
        pltpu.VMEM((TPW, NV), jnp.int32),
    ],
)(_sc_rank_kernel)


def _block_remain_kernel(g_ref, v0, v1, v2, v3, v4, v5, v6, v7,
                         noise_ref, emb_ref, pe_ref, out_ref):
    valid = [v0, v1, v2, v3, v4, v5, v6, v7]
    n = noise_ref[0]
    j_iota = jax.lax.broadcasted_iota(jnp.int32, (1, NV), 1)

    ranks = jnp.zeros((TB, NV), jnp.int32)
    for i in range(NV):
        ni = n[:, i:i + 1]
        lt = (n < ni)
        eq = (n == ni) & (j_iota < i)
        rank_i = jnp.sum((lt | eq).astype(jnp.int32), axis=1, keepdims=True)
        ranks = ranks + rank_i * (j_iota == i).astype(jnp.int32)

    shuffle = jnp.zeros((TB, NV), jnp.int32)
    for i in range(NV):
        ri = ranks[:, i:i + 1]
        shuffle = shuffle + jnp.where(ri == j_iota, i, 0)

    pe = pe_ref[pl.ds((pl.program_id(0) % TBLK) * TB, TB), :]
    out_ref[0, :, 0, :] = g_ref[0] + emb_ref[0:1, :] + pe

    vp = [valid[i][0] + emb_ref[i + 1:i + 2, :] for i in range(NV)]
    for k in range(NR):
        sel = shuffle[:, k:k + 1]
        acc = vp[0]
        for i in range(1, NV):
            acc = jnp.where(sel == i, vp[i], acc)
        out_ref[0, :, k + 1, :] = acc + pe


@jax.jit
def _run(g, vs, noise, emb16, pe):
    masked_f, revert_f = _sc_rank(noise)
    tok_spec = pl.BlockSpec((1, TB, D), lambda i: (i // TBLK, i % TBLK, 0))
    out = pl.pallas_call(
        _block_remain_kernel,
        grid=(B * TBLK,),
        in_specs=[tok_spec] * (1 + NV) + [
            pl.BlockSpec((1, TB, NV), lambda i: (i // TBLK, i % TBLK, 0)),
            pl.BlockSpec((16, D), lambda i: (0, 0)),
            pl.BlockSpec((T, D), lambda i: (0, 0)),
        ],
        out_specs=pl.BlockSpec((1, TB, NR + 1, D), lambda i: (i // TBLK, i % TBLK, 0, 0)),
        out_shape=jax.ShapeDtypeStruct((B, T, NR + 1, D), jnp.float32),
    )(g, *vs, noise, emb16, pe)
    return (out, masked_f, revert_f)


def kernel(data_global, data_t0, data_t1, data_t2, data_t3, data_t4,
           data_t5, data_t6, data_t7, noise, mod_emb):
    vs = [data_t0, data_t1, data_t2, data_t3, data_t4, data_t5, data_t6, data_t7]
    emb16 = jnp.zeros((16, D), jnp.float32).at[:NV + 1].set(mod_emb)
    pe = jnp.asarray(_sinusoidal_pe(T, D))
    return _run(data_global, vs, noise, emb16, pe)
